# baseline (speedup 1.0000x reference)
import functools

import jax
import jax.numpy as jnp
from jax import lax
from jax.experimental import pallas as pl
from jax.experimental.pallas import tpu as pltpu
from jax.experimental.pallas import tpu_sc as plsc

N = 100000
D = 128
B = 4096
NC = 2
NS = 16
NW = NC * NS
CH = 128
NB = 4
NCHUNK = N // CH
TAIL = N - NCHUNK * CH
ITERS = (NCHUNK + NW - 1) // NW
QITERS = (ITERS + 2 + NB - 1) // NB
SEG_PER_TILE = B // NS
HSEG = SEG_PER_TILE // 2


def _sc_pool_body(xq, xa, xc, xt, bq, ba, bc, bt,
                  sums_out, cnts_out,
                  rows4_v, idx4_v, idx_tail_v, zb_v, acc,
                  sem_r0, sem_r1, sem_r2, sem_r3,
                  sem_i0, sem_i1, sem_i2, sem_i3):
    c = lax.axis_index("c")
    s = lax.axis_index("s")
    wid = c * NS + s

    sem_r = (sem_r0, sem_r1, sem_r2, sem_r3)
    sem_i = (sem_i0, sem_i1, sem_i2, sem_i3)

    zeros16 = jnp.zeros((16,), jnp.float32)
    ones16 = jnp.ones((16,), jnp.float32)

    def fill_zb(i, _):
        for k in range(D // 16):
            zb_v[i, pl.ds(k * 16, 16)] = zeros16
        return 0
    lax.fori_loop(0, HSEG, fill_zb, 0)

    def fill_ones(i, _):
        for k in range(D // 16):
            rows4_v[0, i, pl.ds(k * 16, 16)] = ones16
        return 0

    seg_base = s * SEG_PER_TILE

    def zero_acc():
        for h in range(2):
            pltpu.sync_copy(zb_v, acc.at[pl.ds(seg_base + h * HSEG, HSEG), :])
        plsc.subcore_barrier()

    def flush_acc(out, ti):
        plsc.subcore_barrier()
        for h in range(2):
            pltpu.sync_copy(acc.at[pl.ds(seg_base + h * HSEG, HSEG), :], zb_v)
            pltpu.sync_copy(zb_v, out.at[c, ti,
                                         pl.ds(seg_base + h * HSEG, HSEG), :])
        lax.fori_loop(0, HSEG, fill_zb, 0)

    def rows_cp(x_hbm, q, b):
        g = wid + q * NW
        return pltpu.make_async_copy(
            x_hbm.at[pl.ds(g * CH, CH), :], rows4_v.at[b], sem_r[b])

    def idx_cp(b_hbm, q, b):
        g = wid + q * NW
        return pltpu.make_async_copy(
            b_hbm.at[pl.ds(g * CH, CH)], idx4_v.at[b], sem_i[b])


    def in_bounds(q):
        return (wid + q * NW) < NCHUNK

    def run_pass(x_hbm, b_hbm, with_rows):
        def prefetch(q, b):
            if with_rows:
                rows_cp(x_hbm, q, b).start()
            idx_cp(b_hbm, q, b).start()

        def fire(q, b):
            if with_rows:
                rows_cp(x_hbm, q, b).wait()
                idx_cp(b_hbm, q, b).wait()
                pltpu.sync_copy(rows4_v.at[b], acc.at[idx4_v.at[b]], add=True)
            else:
                idx_cp(b_hbm, q, b).wait()
                pltpu.sync_copy(rows4_v.at[0], acc.at[idx4_v.at[b]], add=True)

        for b in range(2):
            @pl.when(in_bounds(b))
            def _():
                prefetch(b, b)

        def body(j, _):
            for b in range(NB):
                q = j * NB + b

                @pl.when(in_bounds(q))
                def _():
                    fire(q, b)

                @pl.when(in_bounds(q + 2))
                def _():
                    prefetch(q + 2, (b + 2) % NB)
            return 0
        lax.fori_loop(0, QITERS, body, 0)

        if TAIL:
            @pl.when(wid == 0)
            def _():
                pltpu.sync_copy(b_hbm.at[pl.ds(NCHUNK * CH, TAIL)], idx_tail_v)
                if with_rows:
                    pltpu.sync_copy(x_hbm.at[pl.ds(NCHUNK * CH, TAIL), :],
                                    rows4_v.at[0, pl.ds(0, TAIL), :])
                    pltpu.sync_copy(rows4_v.at[0, pl.ds(0, TAIL), :],
                                    acc.at[idx_tail_v], add=True)
                else:
                    pltpu.sync_copy(rows4_v.at[0, pl.ds(0, TAIL), :],
                                    acc.at[idx_tail_v], add=True)

    for ti, (x_hbm, b_hbm) in enumerate(((xq, bq), (xa, ba), (xc, bc), (xt, bt))):
        zero_acc()
        run_pass(x_hbm, b_hbm, True)
        flush_acc(sums_out, ti)

        zero_acc()
        lax.fori_loop(0, CH, fill_ones, 0)
        run_pass(x_hbm, b_hbm, False)
        flush_acc(cnts_out, ti)


_sc_pool = functools.partial(
    pl.kernel,
    out_type=(
        jax.ShapeDtypeStruct((NC, 4, B, D), jnp.float32),
        jax.ShapeDtypeStruct((NC, 4, B, D), jnp.float32),
    ),
    mesh=plsc.VectorSubcoreMesh(core_axis_name="c", subcore_axis_name="s",
                                num_cores=NC, num_subcores=NS),
    scratch_types=[
        pltpu.VMEM((NB, CH, D), jnp.float32),
        pltpu.VMEM((NB, CH), jnp.int32),
        pltpu.VMEM((TAIL,), jnp.int32),
        pltpu.VMEM((HSEG, D), jnp.float32),
        pltpu.MemorySpace.VMEM_SHARED((B, D), jnp.float32),
        pltpu.SemaphoreType.DMA,
        pltpu.SemaphoreType.DMA,
        pltpu.SemaphoreType.DMA,
        pltpu.SemaphoreType.DMA,
        pltpu.SemaphoreType.DMA,
        pltpu.SemaphoreType.DMA,
        pltpu.SemaphoreType.DMA,
        pltpu.SemaphoreType.DMA,
    ],
)(_sc_pool_body)


def _mlp_body(sums_ref, cnts_ref, post_ref, w1_ref, b1_ref, w2_ref, b2_ref, o_ref):
    parts = []
    for t in range(4):
        st = sums_ref[0, t] + sums_ref[1, t]
        cnt = cnts_ref[0, t][:, 0:1] + cnts_ref[1, t][:, 0:1]
        parts.append(st / jnp.maximum(cnt, 1.0))
    parts.append(post_ref[...])
    x = jnp.concatenate(parts, axis=1)
    h = jnp.dot(x, w1_ref[...], preferred_element_type=jnp.float32) + b1_ref[...]
    h = jnp.where(h >= 0, h, 0.01 * h)
    o = jnp.dot(h, w2_ref[...], preferred_element_type=jnp.float32) + b2_ref[...]
    o = jnp.where(o >= 0, o, 0.01 * o)
    m = jnp.max(o, axis=-1, keepdims=True)
    e = jnp.exp(o - m)
    o_ref[...] = e / jnp.sum(e, axis=-1, keepdims=True)


def _mlp(sums, cnts, post_emb, W1, b1, W2, b2):
    BM = 256
    grid = (B // BM,)
    return pl.pallas_call(
        _mlp_body,
        grid=grid,
        in_specs=[
            pl.BlockSpec((NC, 4, BM, D), lambda i: (0, 0, i, 0)),
            pl.BlockSpec((NC, 4, BM, D), lambda i: (0, 0, i, 0)),
            pl.BlockSpec((BM, post_emb.shape[1]), lambda i: (i, 0)),
            pl.BlockSpec(W1.shape, lambda i: (0, 0)),
            pl.BlockSpec((1, b1.shape[0]), lambda i: (0, 0)),
            pl.BlockSpec(W2.shape, lambda i: (0, 0)),
            pl.BlockSpec((1, b2.shape[0]), lambda i: (0, 0)),
        ],
        out_specs=pl.BlockSpec((BM, 2), lambda i: (i, 0)),
        out_shape=jax.ShapeDtypeStruct((B, 2), jnp.float32),
    )(sums, cnts, post_emb, W1, b1.reshape(1, -1), W2, b2.reshape(1, -1))


def kernel(x_question, x_answer, x_comment, x_tag,
           batch_question, batch_answer, batch_comment, batch_tag,
           edge_index, post_emb, W1, b1, W2, b2):
    sums, cnts = _sc_pool(x_question, x_answer, x_comment, x_tag,
                          batch_question, batch_answer, batch_comment,
                          batch_tag)
    return _mlp(sums, cnts, post_emb, W1, b1, W2, b2)


# speedup vs baseline: 1.0152x; 1.0152x over previous
import functools

import jax
import jax.numpy as jnp
from jax import lax
from jax.experimental import pallas as pl
from jax.experimental.pallas import tpu as pltpu
from jax.experimental.pallas import tpu_sc as plsc

N = 100000
D = 128
B = 4096
NC = 2
NS = 16
NW = NC * NS
CH = 128
NB = 4
NCHUNK = N // CH
TAIL = N - NCHUNK * CH
ITERS = (NCHUNK + NW - 1) // NW
QITERS = (ITERS + 2 + NB - 1) // NB
SEG_PER_TILE = B // NS
HSEG = SEG_PER_TILE // 2


def _sc_pool_body(xq, xa, xc, xt, bq, ba, bc, bt,
                  sums_out, cnts_out,
                  rows4_v, idx4_v, idx_tail_v, zb_v, acc,
                  sem_r0, sem_r1, sem_r2, sem_r3,
                  sem_i0, sem_i1, sem_i2, sem_i3):
    c = lax.axis_index("c")
    s = lax.axis_index("s")
    wid = c * NS + s

    sem_r = (sem_r0, sem_r1, sem_r2, sem_r3)
    sem_i = (sem_i0, sem_i1, sem_i2, sem_i3)

    zeros16 = jnp.zeros((16,), jnp.float32)
    ones16 = jnp.ones((16,), jnp.float32)

    def fill_zb(i, _):
        for k in range(D // 16):
            zb_v[i, pl.ds(k * 16, 16)] = zeros16
        return 0
    lax.fori_loop(0, HSEG, fill_zb, 0)

    def fill_ones(i, _):
        for k in range(D // 16):
            rows4_v[0, i, pl.ds(k * 16, 16)] = ones16
        return 0

    seg_base = s * SEG_PER_TILE

    def zero_acc():
        for h in range(2):
            pltpu.sync_copy(zb_v, acc.at[pl.ds(seg_base + h * HSEG, HSEG), :])
        plsc.subcore_barrier()

    def flush_acc(out, ti):
        plsc.subcore_barrier()
        pltpu.sync_copy(acc.at[pl.ds(seg_base, SEG_PER_TILE), :],
                        out.at[c, ti, pl.ds(seg_base, SEG_PER_TILE), :])

    def rows_cp(x_hbm, q, b):
        g = wid + q * NW
        return pltpu.make_async_copy(
            x_hbm.at[pl.ds(g * CH, CH), :], rows4_v.at[b], sem_r[b])

    def idx_cp(b_hbm, q, b):
        g = wid + q * NW
        return pltpu.make_async_copy(
            b_hbm.at[pl.ds(g * CH, CH)], idx4_v.at[b], sem_i[b])


    def in_bounds(q):
        return (wid + q * NW) < NCHUNK

    def run_pass(x_hbm, b_hbm, with_rows):
        def prefetch(q, b):
            if with_rows:
                rows_cp(x_hbm, q, b).start()
            idx_cp(b_hbm, q, b).start()

        def fire(q, b):
            if with_rows:
                rows_cp(x_hbm, q, b).wait()
                idx_cp(b_hbm, q, b).wait()
                pltpu.sync_copy(rows4_v.at[b], acc.at[idx4_v.at[b]], add=True)
            else:
                idx_cp(b_hbm, q, b).wait()
                pltpu.sync_copy(rows4_v.at[0], acc.at[idx4_v.at[b]], add=True)

        for b in range(2):
            @pl.when(in_bounds(b))
            def _():
                prefetch(b, b)

        def body(j, _):
            for b in range(NB):
                q = j * NB + b

                @pl.when(in_bounds(q))
                def _():
                    fire(q, b)

                @pl.when(in_bounds(q + 2))
                def _():
                    prefetch(q + 2, (b + 2) % NB)
            return 0
        lax.fori_loop(0, QITERS, body, 0)

        if TAIL:
            @pl.when(wid == 0)
            def _():
                pltpu.sync_copy(b_hbm.at[pl.ds(NCHUNK * CH, TAIL)], idx_tail_v)
                if with_rows:
                    pltpu.sync_copy(x_hbm.at[pl.ds(NCHUNK * CH, TAIL), :],
                                    rows4_v.at[0, pl.ds(0, TAIL), :])
                    pltpu.sync_copy(rows4_v.at[0, pl.ds(0, TAIL), :],
                                    acc.at[idx_tail_v], add=True)
                else:
                    pltpu.sync_copy(rows4_v.at[0, pl.ds(0, TAIL), :],
                                    acc.at[idx_tail_v], add=True)

    for ti, (x_hbm, b_hbm) in enumerate(((xq, bq), (xa, ba), (xc, bc), (xt, bt))):
        zero_acc()
        run_pass(x_hbm, b_hbm, True)
        flush_acc(sums_out, ti)

        zero_acc()
        lax.fori_loop(0, CH, fill_ones, 0)
        run_pass(x_hbm, b_hbm, False)
        flush_acc(cnts_out, ti)


_sc_pool = functools.partial(
    pl.kernel,
    out_type=(
        jax.ShapeDtypeStruct((NC, 4, B, D), jnp.float32),
        jax.ShapeDtypeStruct((NC, 4, B, D), jnp.float32),
    ),
    mesh=plsc.VectorSubcoreMesh(core_axis_name="c", subcore_axis_name="s",
                                num_cores=NC, num_subcores=NS),
    scratch_types=[
        pltpu.VMEM((NB, CH, D), jnp.float32),
        pltpu.VMEM((NB, CH), jnp.int32),
        pltpu.VMEM((TAIL,), jnp.int32),
        pltpu.VMEM((HSEG, D), jnp.float32),
        pltpu.MemorySpace.VMEM_SHARED((B, D), jnp.float32),
        pltpu.SemaphoreType.DMA,
        pltpu.SemaphoreType.DMA,
        pltpu.SemaphoreType.DMA,
        pltpu.SemaphoreType.DMA,
        pltpu.SemaphoreType.DMA,
        pltpu.SemaphoreType.DMA,
        pltpu.SemaphoreType.DMA,
        pltpu.SemaphoreType.DMA,
    ],
)(_sc_pool_body)


def _mlp_body(sums_ref, cnts_ref, post_ref, w1_ref, b1_ref, w2_ref, b2_ref, o_ref):
    parts = []
    for t in range(4):
        st = sums_ref[0, t] + sums_ref[1, t]
        cnt = cnts_ref[0, t][:, 0:1] + cnts_ref[1, t][:, 0:1]
        parts.append(st / jnp.maximum(cnt, 1.0))
    parts.append(post_ref[...])
    x = jnp.concatenate(parts, axis=1)
    h = jnp.dot(x, w1_ref[...], preferred_element_type=jnp.float32) + b1_ref[...]
    h = jnp.where(h >= 0, h, 0.01 * h)
    o = jnp.dot(h, w2_ref[...], preferred_element_type=jnp.float32) + b2_ref[...]
    o = jnp.where(o >= 0, o, 0.01 * o)
    m = jnp.max(o, axis=-1, keepdims=True)
    e = jnp.exp(o - m)
    o_ref[...] = e / jnp.sum(e, axis=-1, keepdims=True)


def _mlp(sums, cnts, post_emb, W1, b1, W2, b2):
    BM = 256
    grid = (B // BM,)
    return pl.pallas_call(
        _mlp_body,
        grid=grid,
        in_specs=[
            pl.BlockSpec((NC, 4, BM, D), lambda i: (0, 0, i, 0)),
            pl.BlockSpec((NC, 4, BM, D), lambda i: (0, 0, i, 0)),
            pl.BlockSpec((BM, post_emb.shape[1]), lambda i: (i, 0)),
            pl.BlockSpec(W1.shape, lambda i: (0, 0)),
            pl.BlockSpec((1, b1.shape[0]), lambda i: (0, 0)),
            pl.BlockSpec(W2.shape, lambda i: (0, 0)),
            pl.BlockSpec((1, b2.shape[0]), lambda i: (0, 0)),
        ],
        out_specs=pl.BlockSpec((BM, 2), lambda i: (i, 0)),
        out_shape=jax.ShapeDtypeStruct((B, 2), jnp.float32),
    )(sums, cnts, post_emb, W1, b1.reshape(1, -1), W2, b2.reshape(1, -1))


def kernel(x_question, x_answer, x_comment, x_tag,
           batch_question, batch_answer, batch_comment, batch_tag,
           edge_index, post_emb, W1, b1, W2, b2):
    sums, cnts = _sc_pool(x_question, x_answer, x_comment, x_tag,
                          batch_question, batch_answer, batch_comment,
                          batch_tag)
    return _mlp(sums, cnts, post_emb, W1, b1, W2, b2)
